# baseline (speedup 1.0000x reference)
import functools

import jax
import jax.numpy as jnp
from jax import lax
from jax.experimental import pallas as pl
from jax.experimental.pallas import tpu as pltpu
from jax.experimental.pallas import tpu_sc as plsc

NC, NS = 2, 16
NW = NC * NS
CH = 128
IC = 32
OC = 32
KH = 16
CW = 16


def _sc_mesh():
    return plsc.VectorSubcoreMesh(core_axis_name="c", subcore_axis_name="s")


_SC_PARAMS = pltpu.CompilerParams(use_tc_tiling_on_sc=False)


def _gather_rows(table, idx2, c0, nct):
    E = nct * CH
    maxc = (nct + NW - 1) // NW

    @functools.partial(
        pl.kernel,
        out_type=jax.ShapeDtypeStruct((E, IC), jnp.float32),
        mesh=_sc_mesh(),
        compiler_params=_SC_PARAMS,
        scratch_types=[
            pltpu.VMEM((maxc, CH), jnp.int32),
            pltpu.VMEM((2, CH, IC), jnp.float32),
            pltpu.SemaphoreType.DMA,
            pltpu.SemaphoreType.DMA,
        ],
    )
    def k(table_hbm, idx_hbm, out_hbm, idx_v, rows_v, sem_i, sem_g):
        wid = lax.axis_index("s") * NC + lax.axis_index("c")
        nch = (nct - 1 - wid) // NW + 1

        def fire_idx(j, c):
            pltpu.async_copy(idx_hbm.at[c0 + wid + j * NW], idx_v.at[j],
                             sem_i)
            return c
        lax.fori_loop(0, nch, fire_idx, 0)

        def drain_idx(j, c):
            pltpu.make_async_copy(idx_hbm.at[c0], idx_v.at[0], sem_i).wait()
            return c
        lax.fori_loop(0, nch, drain_idx, 0)

        def fire_gather(j):
            pltpu.async_copy(table_hbm.at[idx_v.at[j]],
                             rows_v.at[lax.rem(j, 2)], sem_g)

        fire_gather(0)

        def body(j, c):
            @pl.when(j + 1 < nch)
            def _():
                fire_gather(j + 1)
            pltpu.make_async_copy(table_hbm.at[idx_v.at[j]],
                                  rows_v.at[lax.rem(j, 2)], sem_g).wait()
            base = (wid + j * NW) * CH
            pltpu.sync_copy(rows_v.at[lax.rem(j, 2)],
                            out_hbm.at[pl.ds(base, CH)])
            return c
        lax.fori_loop(0, nch, body, 0)

    return k(table, idx2)


def _scatter_add(msg, dst2, c0, nct, n, zeros_nc):
    rows_per_tile = n // NS
    maxc = (nct + NW - 1) // NW

    out_type = [jax.ShapeDtypeStruct((NC, n, OC), jnp.float32)]
    scratch = [
        pltpu.VMEM((maxc, CH), jnp.int32),
        pltpu.VMEM((2, CH, OC), jnp.float32),
        pltpu.VMEM_SHARED((n, OC), jnp.float32),
        pltpu.SemaphoreType.DMA,
        pltpu.SemaphoreType.DMA,
    ]

    @functools.partial(
        pl.kernel,
        out_type=tuple(out_type),
        mesh=_sc_mesh(),
        compiler_params=_SC_PARAMS,
        scratch_types=scratch,
    )
    def k(msg_hbm, dst_hbm, z_hbm, *refs):
        (sum_out, idx_v, rows_v, acc_s, sem_i, sem_m) = refs
        cid = lax.axis_index("c")
        sid = lax.axis_index("s")
        wid = sid * NC + cid
        nch = (nct - 1 - wid) // NW + 1

        def fire_idx(j, c):
            pltpu.async_copy(dst_hbm.at[c0 + wid + j * NW], idx_v.at[j],
                             sem_i)
            return c
        lax.fori_loop(0, nch, fire_idx, 0)

        @pl.when(sid == 0)
        def _():
            pltpu.sync_copy(z_hbm, acc_s)

        def drain_idx(j, c):
            pltpu.make_async_copy(dst_hbm.at[c0], idx_v.at[0], sem_i).wait()
            return c
        lax.fori_loop(0, nch, drain_idx, 0)
        plsc.subcore_barrier()

        def fire_msg(j):
            base = (wid + j * NW) * CH
            pltpu.async_copy(msg_hbm.at[pl.ds(base, CH)],
                             rows_v.at[lax.rem(j, 2)], sem_m)

        fire_msg(0)

        def body(j, carry):
            @pl.when(j + 1 < nch)
            def _():
                fire_msg(j + 1)
            base = (wid + j * NW) * CH
            pltpu.make_async_copy(msg_hbm.at[pl.ds(base, CH)],
                                  rows_v.at[lax.rem(j, 2)], sem_m).wait()
            pltpu.sync_copy(rows_v.at[lax.rem(j, 2)], acc_s.at[idx_v.at[j]],
                            add=True)
            return carry

        lax.fori_loop(0, nch, body, 0)
        plsc.subcore_barrier()

        r0 = sid * rows_per_tile
        pltpu.sync_copy(acc_s.at[pl.ds(r0, rows_per_tile)],
                        sum_out.at[cid].at[pl.ds(r0, rows_per_tile)])

    return k(msg, dst2, zeros_nc)[0]


def _count_edges(dst2, n, zeros_cnt):
    n_chunks = dst2.shape[0]
    rows_per_tile = n // NS
    maxc = (n_chunks + NW - 1) // NW

    @functools.partial(
        pl.kernel,
        out_type=jax.ShapeDtypeStruct((NC, n, CW), jnp.float32),
        mesh=_sc_mesh(),
        compiler_params=_SC_PARAMS,
        scratch_types=[
            pltpu.VMEM((maxc, CH), jnp.int32),
            pltpu.VMEM((CH, CW), jnp.float32),
            pltpu.VMEM_SHARED((n, CW), jnp.float32),
            pltpu.SemaphoreType.DMA,
        ],
    )
    def k(dst_hbm, zc_hbm, cnt_out, idx_v, ones_v, cnt_s, sem_i):
        cid = lax.axis_index("c")
        sid = lax.axis_index("s")
        wid = sid * NC + cid
        nch = (n_chunks - 1 - wid) // NW + 1

        def fire_idx(j, c):
            pltpu.async_copy(dst_hbm.at[wid + j * NW], idx_v.at[j], sem_i)
            return c
        lax.fori_loop(0, nch, fire_idx, 0)

        @pl.when(sid == 0)
        def _():
            pltpu.sync_copy(zc_hbm, cnt_s)

        def fill(i, c):
            ones_v[i, :] = jnp.ones((CW,), jnp.float32)
            return c
        lax.fori_loop(0, CH, fill, 0)

        def drain_idx(j, c):
            pltpu.make_async_copy(dst_hbm.at[wid], idx_v.at[0], sem_i).wait()
            return c
        lax.fori_loop(0, nch, drain_idx, 0)
        plsc.subcore_barrier()

        def body(j, c):
            pltpu.sync_copy(ones_v, cnt_s.at[idx_v.at[j]], add=True)
            return c
        lax.fori_loop(0, nch, body, 0)
        plsc.subcore_barrier()

        r0 = sid * rows_per_tile
        pltpu.sync_copy(cnt_s.at[pl.ds(r0, rows_per_tile)],
                        cnt_out.at[cid].at[pl.ds(r0, rows_per_tile)])

    return k(dst2, zeros_cnt)


def _edge_hidden(ea4, w1bd1, b1t1, w1bd2, b1t2):
    rows = ea4.shape[0]
    RB = 4000
    grid = rows // RB

    def body(ea_ref, w1_ref, b1_ref, w2_ref, b2_ref, o_ref):
        ea = ea_ref[...]
        h1 = jnp.maximum(
            jnp.dot(ea, w1_ref[...], preferred_element_type=jnp.float32)
            + b1_ref[...], 0.0)
        h2 = jnp.maximum(
            jnp.dot(ea, w2_ref[...], preferred_element_type=jnp.float32)
            + b2_ref[...], 0.0)
        o_ref[...] = jnp.concatenate([h1, h2], axis=1)

    full = lambda s: pl.BlockSpec(s, lambda i: (0, 0))
    return pl.pallas_call(
        body,
        grid=(grid,),
        in_specs=[
            pl.BlockSpec((RB, 16), lambda i: (i, 0)),
            full((16, 4 * KH)), full((1, 4 * KH)),
            full((16, 4 * KH)), full((1, 4 * KH)),
        ],
        out_specs=pl.BlockSpec((RB, 8 * KH), lambda i: (i, 0)),
        out_shape=jax.ShapeDtypeStruct((rows, 8 * KH), jnp.float32),
    )(ea4, w1bd1, b1t1, w1bd2, b1t2)


def _edge_messages(hcombo, xj128, layer, roff, R, W2r, B2r):
    rows = xj128.shape[0]
    RB = 2000
    grid = rows // RB
    hoff = layer * 4 * KH

    def body(h_ref, xj_ref, R_ref, W2_ref, B2_ref, o_ref):
        xj = xj_ref[...]
        parts = []
        for j in range(4):
            h_j = h_ref[:, hoff + j * KH:hoff + (j + 1) * KH]
            x_j = xj[:, j * IC:(j + 1) * IC]
            A = jnp.dot(h_j, R_ref[...], preferred_element_type=jnp.float32)
            B = jnp.tile(x_j, (1, KH))
            acc = jnp.dot((A * B).astype(jnp.bfloat16), W2_ref[...],
                          preferred_element_type=jnp.float32)
            acc = acc + jnp.dot(x_j, B2_ref[...],
                                preferred_element_type=jnp.float32)
            parts.append(acc)
        o_ref[...] = jnp.concatenate(parts, axis=1)

    full = lambda s: pl.BlockSpec(s, lambda i: (0, 0))
    return pl.pallas_call(
        body,
        grid=(grid,),
        in_specs=[
            pl.BlockSpec((RB, 8 * KH), lambda i: (i + roff, 0)),
            pl.BlockSpec((RB, 4 * IC), lambda i: (i, 0)),
            full((KH, KH * IC)),
            full((KH * IC, OC)), full((IC, OC)),
        ],
        out_specs=pl.BlockSpec((RB, 4 * OC), lambda i: (i, 0)),
        out_shape=jax.ShapeDtypeStruct((rows, 4 * OC), jnp.float32),
    )(hcombo, xj128, R, W2r.astype(jnp.bfloat16), B2r)


def _combine1(sums_a, sums_b, cnts, x, root1, b1r):
    n = x.shape[0]

    def body(sa_ref, sb_ref, c_ref, x_ref, r_ref, b_ref, o_ref, inv_ref):
        cnt = c_ref[0] + c_ref[1]
        inv = 1.0 / jnp.maximum(cnt[:, 0:1], 1.0)
        mean = (sa_ref[0] + sa_ref[1] + sb_ref[0] + sb_ref[1]) * inv
        o_ref[...] = jnp.maximum(
            mean + jnp.dot(x_ref[...], r_ref[...],
                           preferred_element_type=jnp.float32) + b_ref[...],
            0.0)
        inv_ref[...] = inv

    return pl.pallas_call(
        body,
        out_shape=(jax.ShapeDtypeStruct((n, OC), jnp.float32),
                   jax.ShapeDtypeStruct((n, 1), jnp.float32)),
    )(sums_a, sums_b, cnts, x, root1, b1r)


def _combine2(sums_a, sums_b, inv, h1, root2, b2r, fc1_w, fc1_br, fc2_w,
              fc2_br):
    n = h1.shape[0]

    def body(sa_ref, sb_ref, i_ref, h_ref, r_ref, b_ref, f1_ref, f1b_ref,
             f2_ref, f2b_ref, o_ref):
        mean = (sa_ref[0] + sa_ref[1] + sb_ref[0] + sb_ref[1]) * i_ref[...]
        h2 = jnp.maximum(
            mean + jnp.dot(h_ref[...], r_ref[...],
                           preferred_element_type=jnp.float32) + b_ref[...],
            0.0)
        h3 = jnp.maximum(
            jnp.dot(h2, f1_ref[...], preferred_element_type=jnp.float32)
            + f1b_ref[...], 0.0)
        o_ref[...] = (jnp.dot(h3, f2_ref[...],
                              preferred_element_type=jnp.float32)
                      + f2b_ref[...])

    return pl.pallas_call(
        body,
        out_shape=jax.ShapeDtypeStruct((n, 1), jnp.float32),
    )(sums_a, sums_b, inv, h1, root2, b2r, fc1_w, fc1_br, fc2_w, fc2_br)


def _prep_w2(w2, b2, in_ch):
    W = w2.reshape(KH, in_ch, OC)
    B = b2.reshape(in_ch, OC)
    if in_ch < IC:
        W = jnp.pad(W, ((0, 0), (0, IC - in_ch), (0, 0)))
        B = jnp.pad(B, ((0, IC - in_ch), (0, 0)))
    return W.reshape(KH * IC, OC), B


def kernel(x, edge_index, edge_attr, nn1_w1, nn1_b1, nn1_w2, nn1_b2, root1,
           bias1, nn2_w1, nn2_b1, nn2_w2, nn2_b2, root2, bias2, fc1_w, fc1_b,
           fc2_w, fc2_b):
    n = x.shape[0]
    src = edge_index[0].astype(jnp.int32)
    dst = edge_index[1].astype(jnp.int32)

    R = jnp.repeat(jnp.eye(KH, dtype=jnp.float32), IC, axis=1)
    W2r1, B2r1 = _prep_w2(nn1_w2, nn1_b2, 3)
    W2r2, B2r2 = _prep_w2(nn2_w2, nn2_b2, IC)

    x_pad = jnp.pad(x, ((0, 0), (0, IC - x.shape[1])))
    zeros_nc = jnp.zeros((n, OC), jnp.float32)
    zeros_cnt = jnp.zeros((n, CW), jnp.float32)

    E = src.shape[0]
    src2 = src.reshape(E // CH, CH)
    dst2 = dst.reshape(E // CH, CH)

    ea4 = edge_attr.reshape(E // 4, 16)
    zblk = jnp.zeros((4, KH), jnp.float32)
    w1bd1 = jnp.block([[nn1_w1 if i == j else zblk for j in range(4)]
                       for i in range(4)])
    w1bd2 = jnp.block([[nn2_w1 if i == j else zblk for j in range(4)]
                       for i in range(4)])
    hcombo = _edge_hidden(ea4, w1bd1, jnp.tile(nn1_b1, 4).reshape(1, 4 * KH),
                          w1bd2, jnp.tile(nn2_b1, 4).reshape(1, 4 * KH))

    NCHK = E // CH
    HC = NCHK // 2
    EH = HC * CH
    RBH = EH // (4 * 2000)

    cnts = _count_edges(dst2, n, zeros_cnt)

    def layer(table, layer_idx, W2r, B2r):
        sums = []
        for half in range(2):
            xj = _gather_rows(table, src2, half * HC, HC)
            msg = _edge_messages(hcombo, xj.reshape(EH // 4, 4 * IC),
                                 layer_idx, half * RBH, R, W2r, B2r)
            sums.append(_scatter_add(msg.reshape(EH, OC), dst2, half * HC,
                                     HC, n, zeros_nc))
        return sums

    s1a, s1b = layer(x_pad, 0, W2r1, B2r1)
    h1, inv_cnt = _combine1(s1a, s1b, cnts, x, root1, bias1.reshape(1, OC))
    s2a, s2b = layer(h1, 1, W2r2, B2r2)
    out = _combine2(s2a, s2b, inv_cnt, h1, root2, bias2.reshape(1, OC),
                    fc1_w, fc1_b.reshape(1, OC), fc2_w, fc2_b.reshape(1, 1))
    return out


# speedup vs baseline: 1.0059x; 1.0059x over previous
import functools

import jax
import jax.numpy as jnp
from jax import lax
from jax.experimental import pallas as pl
from jax.experimental.pallas import tpu as pltpu
from jax.experimental.pallas import tpu_sc as plsc

NC, NS = 2, 16
NW = NC * NS
CH = 128
IC = 32
OC = 32
KH = 16
CW = 16


def _sc_mesh():
    return plsc.VectorSubcoreMesh(core_axis_name="c", subcore_axis_name="s")


_SC_PARAMS = pltpu.CompilerParams(use_tc_tiling_on_sc=False)


def _gather_rows(table, idx2, c0, nct):
    E = nct * CH
    maxc = (nct + NW - 1) // NW

    @functools.partial(
        pl.kernel,
        out_type=jax.ShapeDtypeStruct((E, IC), jnp.float32),
        mesh=_sc_mesh(),
        compiler_params=_SC_PARAMS,
        scratch_types=[
            pltpu.VMEM((maxc, CH), jnp.int32),
            pltpu.VMEM((2, CH, IC), jnp.float32),
            pltpu.SemaphoreType.DMA,
            pltpu.SemaphoreType.DMA,
        ],
    )
    def k(table_hbm, idx_hbm, out_hbm, idx_v, rows_v, sem_i, sem_g):
        wid = lax.axis_index("s") * NC + lax.axis_index("c")
        nch = (nct - 1 - wid) // NW + 1

        def fire_idx(j, c):
            pltpu.async_copy(idx_hbm.at[c0 + wid + j * NW], idx_v.at[j],
                             sem_i)
            return c
        lax.fori_loop(0, nch, fire_idx, 0)

        def drain_idx(j, c):
            pltpu.make_async_copy(idx_hbm.at[c0], idx_v.at[0], sem_i).wait()
            return c
        lax.fori_loop(0, nch, drain_idx, 0)

        def fire_gather(j):
            pltpu.async_copy(table_hbm.at[idx_v.at[j]],
                             rows_v.at[lax.rem(j, 2)], sem_g)

        fire_gather(0)

        def body(j, c):
            @pl.when(j + 1 < nch)
            def _():
                fire_gather(j + 1)
            pltpu.make_async_copy(table_hbm.at[idx_v.at[j]],
                                  rows_v.at[lax.rem(j, 2)], sem_g).wait()
            base = (wid + j * NW) * CH
            pltpu.sync_copy(rows_v.at[lax.rem(j, 2)],
                            out_hbm.at[pl.ds(base, CH)])
            return c
        lax.fori_loop(0, nch, body, 0)

    return k(table, idx2)


def _scatter_add(msg, dst2, c0, nct, n, zeros_nc):
    rows_per_tile = n // NS
    maxc = (nct + NW - 1) // NW

    out_type = [jax.ShapeDtypeStruct((NC, n, OC), jnp.float32)]
    scratch = [
        pltpu.VMEM((maxc, CH), jnp.int32),
        pltpu.VMEM((2, CH, OC), jnp.float32),
        pltpu.VMEM_SHARED((n, OC), jnp.float32),
        pltpu.SemaphoreType.DMA,
        pltpu.SemaphoreType.DMA,
    ]

    @functools.partial(
        pl.kernel,
        out_type=tuple(out_type),
        mesh=_sc_mesh(),
        compiler_params=_SC_PARAMS,
        scratch_types=scratch,
    )
    def k(msg_hbm, dst_hbm, z_hbm, *refs):
        (sum_out, idx_v, rows_v, acc_s, sem_i, sem_m) = refs
        cid = lax.axis_index("c")
        sid = lax.axis_index("s")
        wid = sid * NC + cid
        nch = (nct - 1 - wid) // NW + 1

        def fire_idx(j, c):
            pltpu.async_copy(dst_hbm.at[c0 + wid + j * NW], idx_v.at[j],
                             sem_i)
            return c
        lax.fori_loop(0, nch, fire_idx, 0)

        @pl.when(sid == 0)
        def _():
            pltpu.sync_copy(z_hbm, acc_s)

        def drain_idx(j, c):
            pltpu.make_async_copy(dst_hbm.at[c0], idx_v.at[0], sem_i).wait()
            return c
        lax.fori_loop(0, nch, drain_idx, 0)
        plsc.subcore_barrier()

        def fire_msg(j):
            base = (wid + j * NW) * CH
            pltpu.async_copy(msg_hbm.at[pl.ds(base, CH)],
                             rows_v.at[lax.rem(j, 2)], sem_m)

        fire_msg(0)

        def body(j, carry):
            @pl.when(j + 1 < nch)
            def _():
                fire_msg(j + 1)
            base = (wid + j * NW) * CH
            pltpu.make_async_copy(msg_hbm.at[pl.ds(base, CH)],
                                  rows_v.at[lax.rem(j, 2)], sem_m).wait()
            pltpu.sync_copy(rows_v.at[lax.rem(j, 2)], acc_s.at[idx_v.at[j]],
                            add=True)
            return carry

        lax.fori_loop(0, nch, body, 0)
        plsc.subcore_barrier()

        r0 = sid * rows_per_tile
        pltpu.sync_copy(acc_s.at[pl.ds(r0, rows_per_tile)],
                        sum_out.at[cid].at[pl.ds(r0, rows_per_tile)])

    return k(msg, dst2, zeros_nc)[0]


def _count_edges(dst2, n, zeros_cnt):
    n_chunks = dst2.shape[0]
    rows_per_tile = n // NS
    maxc = (n_chunks + NW - 1) // NW

    @functools.partial(
        pl.kernel,
        out_type=jax.ShapeDtypeStruct((NC, n, CW), jnp.float32),
        mesh=_sc_mesh(),
        compiler_params=_SC_PARAMS,
        scratch_types=[
            pltpu.VMEM((maxc, CH), jnp.int32),
            pltpu.VMEM((CH, CW), jnp.float32),
            pltpu.VMEM_SHARED((n, CW), jnp.float32),
            pltpu.SemaphoreType.DMA,
        ],
    )
    def k(dst_hbm, zc_hbm, cnt_out, idx_v, ones_v, cnt_s, sem_i):
        cid = lax.axis_index("c")
        sid = lax.axis_index("s")
        wid = sid * NC + cid
        nch = (n_chunks - 1 - wid) // NW + 1

        def fire_idx(j, c):
            pltpu.async_copy(dst_hbm.at[wid + j * NW], idx_v.at[j], sem_i)
            return c
        lax.fori_loop(0, nch, fire_idx, 0)

        @pl.when(sid == 0)
        def _():
            pltpu.sync_copy(zc_hbm, cnt_s)

        def fill(i, c):
            ones_v[i, :] = jnp.ones((CW,), jnp.float32)
            return c
        lax.fori_loop(0, CH, fill, 0)

        def drain_idx(j, c):
            pltpu.make_async_copy(dst_hbm.at[wid], idx_v.at[0], sem_i).wait()
            return c
        lax.fori_loop(0, nch, drain_idx, 0)
        plsc.subcore_barrier()

        def body(j, c):
            pltpu.sync_copy(ones_v, cnt_s.at[idx_v.at[j]], add=True)
            return c
        lax.fori_loop(0, nch, body, 0)
        plsc.subcore_barrier()

        r0 = sid * rows_per_tile
        pltpu.sync_copy(cnt_s.at[pl.ds(r0, rows_per_tile)],
                        cnt_out.at[cid].at[pl.ds(r0, rows_per_tile)])

    return k(dst2, zeros_cnt)


def _edge_hidden(ea4, w1bd1, b1t1, w1bd2, b1t2):
    rows = ea4.shape[0]
    RB = 4000
    grid = rows // RB

    def body(ea_ref, w1_ref, b1_ref, w2_ref, b2_ref, o_ref):
        ea = ea_ref[...]
        h1 = jnp.maximum(
            jnp.dot(ea, w1_ref[...], preferred_element_type=jnp.float32)
            + b1_ref[...], 0.0)
        h2 = jnp.maximum(
            jnp.dot(ea, w2_ref[...], preferred_element_type=jnp.float32)
            + b2_ref[...], 0.0)
        o_ref[...] = jnp.concatenate([h1, h2], axis=1)

    full = lambda s: pl.BlockSpec(s, lambda i: (0, 0))
    return pl.pallas_call(
        body,
        grid=(grid,),
        in_specs=[
            pl.BlockSpec((RB, 16), lambda i: (i, 0)),
            full((16, 4 * KH)), full((1, 4 * KH)),
            full((16, 4 * KH)), full((1, 4 * KH)),
        ],
        out_specs=pl.BlockSpec((RB, 8 * KH), lambda i: (i, 0)),
        out_shape=jax.ShapeDtypeStruct((rows, 8 * KH), jnp.float32),
    )(ea4, w1bd1, b1t1, w1bd2, b1t2)


def _edge_messages(hcombo, xj128, layer, roff, R, W2r, B2r):
    rows = xj128.shape[0]
    RB = 4000
    grid = rows // RB
    hoff = layer * 4 * KH

    def body(h_ref, xj_ref, R_ref, W2_ref, B2_ref, o_ref):
        xj = xj_ref[...]
        parts = []
        for j in range(4):
            h_j = h_ref[:, hoff + j * KH:hoff + (j + 1) * KH]
            x_j = xj[:, j * IC:(j + 1) * IC]
            A = jnp.dot(h_j, R_ref[...], preferred_element_type=jnp.float32)
            B = jnp.tile(x_j, (1, KH))
            acc = jnp.dot(A * B, W2_ref[...],
                          preferred_element_type=jnp.float32)
            acc = acc + jnp.dot(x_j, B2_ref[...],
                                preferred_element_type=jnp.float32)
            parts.append(acc)
        o_ref[...] = jnp.concatenate(parts, axis=1)

    full = lambda s: pl.BlockSpec(s, lambda i: (0, 0))
    return pl.pallas_call(
        body,
        grid=(grid,),
        in_specs=[
            pl.BlockSpec((RB, 8 * KH), lambda i: (i + roff, 0)),
            pl.BlockSpec((RB, 4 * IC), lambda i: (i, 0)),
            full((KH, KH * IC)),
            full((KH * IC, OC)), full((IC, OC)),
        ],
        out_specs=pl.BlockSpec((RB, 4 * OC), lambda i: (i, 0)),
        out_shape=jax.ShapeDtypeStruct((rows, 4 * OC), jnp.float32),
    )(hcombo, xj128, R, W2r, B2r)


def _combine1(sums_a, sums_b, cnts, x, root1, b1r):
    n = x.shape[0]

    def body(sa_ref, sb_ref, c_ref, x_ref, r_ref, b_ref, o_ref, inv_ref):
        cnt = c_ref[0] + c_ref[1]
        inv = 1.0 / jnp.maximum(cnt[:, 0:1], 1.0)
        mean = (sa_ref[0] + sa_ref[1] + sb_ref[0] + sb_ref[1]) * inv
        o_ref[...] = jnp.maximum(
            mean + jnp.dot(x_ref[...], r_ref[...],
                           preferred_element_type=jnp.float32) + b_ref[...],
            0.0)
        inv_ref[...] = inv

    return pl.pallas_call(
        body,
        out_shape=(jax.ShapeDtypeStruct((n, OC), jnp.float32),
                   jax.ShapeDtypeStruct((n, 1), jnp.float32)),
    )(sums_a, sums_b, cnts, x, root1, b1r)


def _combine2(sums_a, sums_b, inv, h1, root2, b2r, fc1_w, fc1_br, fc2_w,
              fc2_br):
    n = h1.shape[0]

    def body(sa_ref, sb_ref, i_ref, h_ref, r_ref, b_ref, f1_ref, f1b_ref,
             f2_ref, f2b_ref, o_ref):
        mean = (sa_ref[0] + sa_ref[1] + sb_ref[0] + sb_ref[1]) * i_ref[...]
        h2 = jnp.maximum(
            mean + jnp.dot(h_ref[...], r_ref[...],
                           preferred_element_type=jnp.float32) + b_ref[...],
            0.0)
        h3 = jnp.maximum(
            jnp.dot(h2, f1_ref[...], preferred_element_type=jnp.float32)
            + f1b_ref[...], 0.0)
        o_ref[...] = (jnp.dot(h3, f2_ref[...],
                              preferred_element_type=jnp.float32)
                      + f2b_ref[...])

    return pl.pallas_call(
        body,
        out_shape=jax.ShapeDtypeStruct((n, 1), jnp.float32),
    )(sums_a, sums_b, inv, h1, root2, b2r, fc1_w, fc1_br, fc2_w, fc2_br)


def _prep_w2(w2, b2, in_ch):
    W = w2.reshape(KH, in_ch, OC)
    B = b2.reshape(in_ch, OC)
    if in_ch < IC:
        W = jnp.pad(W, ((0, 0), (0, IC - in_ch), (0, 0)))
        B = jnp.pad(B, ((0, IC - in_ch), (0, 0)))
    return W.reshape(KH * IC, OC), B


def kernel(x, edge_index, edge_attr, nn1_w1, nn1_b1, nn1_w2, nn1_b2, root1,
           bias1, nn2_w1, nn2_b1, nn2_w2, nn2_b2, root2, bias2, fc1_w, fc1_b,
           fc2_w, fc2_b):
    n = x.shape[0]
    src = edge_index[0].astype(jnp.int32)
    dst = edge_index[1].astype(jnp.int32)

    R = jnp.repeat(jnp.eye(KH, dtype=jnp.float32), IC, axis=1)
    W2r1, B2r1 = _prep_w2(nn1_w2, nn1_b2, 3)
    W2r2, B2r2 = _prep_w2(nn2_w2, nn2_b2, IC)

    x_pad = jnp.pad(x, ((0, 0), (0, IC - x.shape[1])))
    zeros_nc = jnp.zeros((n, OC), jnp.float32)
    zeros_cnt = jnp.zeros((n, CW), jnp.float32)

    E = src.shape[0]
    src2 = src.reshape(E // CH, CH)
    dst2 = dst.reshape(E // CH, CH)

    ea4 = edge_attr.reshape(E // 4, 16)
    zblk = jnp.zeros((4, KH), jnp.float32)
    w1bd1 = jnp.block([[nn1_w1 if i == j else zblk for j in range(4)]
                       for i in range(4)])
    w1bd2 = jnp.block([[nn2_w1 if i == j else zblk for j in range(4)]
                       for i in range(4)])
    hcombo = _edge_hidden(ea4, w1bd1, jnp.tile(nn1_b1, 4).reshape(1, 4 * KH),
                          w1bd2, jnp.tile(nn2_b1, 4).reshape(1, 4 * KH))

    NCHK = E // CH
    HC = NCHK // 2
    EH = HC * CH
    RBH = EH // (4 * 4000)

    cnts = _count_edges(dst2, n, zeros_cnt)

    def layer(table, layer_idx, W2r, B2r):
        sums = []
        for half in range(2):
            xj = _gather_rows(table, src2, half * HC, HC)
            msg = _edge_messages(hcombo, xj.reshape(EH // 4, 4 * IC),
                                 layer_idx, half * RBH, R, W2r, B2r)
            sums.append(_scatter_add(msg.reshape(EH, OC), dst2, half * HC,
                                     HC, n, zeros_nc))
        return sums

    s1a, s1b = layer(x_pad, 0, W2r1, B2r1)
    h1, inv_cnt = _combine1(s1a, s1b, cnts, x, root1, bias1.reshape(1, OC))
    s2a, s2b = layer(h1, 1, W2r2, B2r2)
    out = _combine2(s2a, s2b, inv_cnt, h1, root2, bias2.reshape(1, OC),
                    fc1_w, fc1_b.reshape(1, OC), fc2_w, fc2_b.reshape(1, 1))
    return out


# speedup vs baseline: 1.0085x; 1.0026x over previous
import functools

import jax
import jax.numpy as jnp
from jax import lax
from jax.experimental import pallas as pl
from jax.experimental.pallas import tpu as pltpu
from jax.experimental.pallas import tpu_sc as plsc

NC, NS = 2, 16
NW = NC * NS
CH = 128
IC = 32
OC = 32
KH = 16
CW = 16


def _sc_mesh():
    return plsc.VectorSubcoreMesh(core_axis_name="c", subcore_axis_name="s")


_SC_PARAMS = pltpu.CompilerParams(use_tc_tiling_on_sc=False)


def _gather_rows(table, idx2, c0, nct):
    E = nct * CH
    maxc = (nct + NW - 1) // NW

    @functools.partial(
        pl.kernel,
        out_type=jax.ShapeDtypeStruct((E, IC), jnp.float32),
        mesh=_sc_mesh(),
        compiler_params=_SC_PARAMS,
        scratch_types=[
            pltpu.VMEM((maxc, CH), jnp.int32),
            pltpu.VMEM((3, CH, IC), jnp.float32),
            pltpu.SemaphoreType.DMA,
            pltpu.SemaphoreType.DMA,
            pltpu.SemaphoreType.DMA,
        ],
    )
    def k(table_hbm, idx_hbm, out_hbm, idx_v, rows_v, sem_i, sem_g, sem_w):
        wid = lax.axis_index("s") * NC + lax.axis_index("c")
        nch = (nct - 1 - wid) // NW + 1

        def fire_idx(j, c):
            pltpu.async_copy(idx_hbm.at[c0 + wid + j * NW], idx_v.at[j],
                             sem_i)
            return c
        lax.fori_loop(0, nch, fire_idx, 0)

        def drain_idx(j, c):
            pltpu.make_async_copy(idx_hbm.at[c0], idx_v.at[0], sem_i).wait()
            return c
        lax.fori_loop(0, nch, drain_idx, 0)

        def fire_gather(j):
            pltpu.async_copy(table_hbm.at[idx_v.at[j]],
                             rows_v.at[lax.rem(j, 3)], sem_g)

        def wb_slices(j):
            base = (wid + j * NW) * CH
            return rows_v.at[lax.rem(j, 3)], out_hbm.at[pl.ds(base, CH)]

        fire_gather(0)

        @pl.when(1 < nch)
        def _():
            fire_gather(1)

        def body(j, c):
            @pl.when(j + 2 < nch)
            def _():
                @pl.when(j >= 1)
                def _():
                    src, dstp = wb_slices(j - 1)
                    pltpu.make_async_copy(src, dstp, sem_w).wait()
                fire_gather(j + 2)
            pltpu.make_async_copy(table_hbm.at[idx_v.at[j]],
                                  rows_v.at[lax.rem(j, 3)], sem_g).wait()
            src, dstp = wb_slices(j)
            pltpu.async_copy(src, dstp, sem_w)
            return c
        lax.fori_loop(0, nch, body, 0)

        def drain_wb(j, c):
            src, dstp = wb_slices(j)
            pltpu.make_async_copy(src, dstp, sem_w).wait()
            return c
        lax.fori_loop(jnp.maximum(nch - 3, 0), nch, drain_wb, 0)

    return k(table, idx2)


def _scatter_add(msg, dst2, c0, nct, n, zeros_nc):
    rows_per_tile = n // NS
    maxc = (nct + NW - 1) // NW

    out_type = [jax.ShapeDtypeStruct((NC, n, OC), jnp.float32)]
    scratch = [
        pltpu.VMEM((maxc, CH), jnp.int32),
        pltpu.VMEM((2, CH, OC), jnp.float32),
        pltpu.VMEM_SHARED((n, OC), jnp.float32),
        pltpu.SemaphoreType.DMA,
        pltpu.SemaphoreType.DMA,
        pltpu.SemaphoreType.DMA,
    ]

    @functools.partial(
        pl.kernel,
        out_type=tuple(out_type),
        mesh=_sc_mesh(),
        compiler_params=_SC_PARAMS,
        scratch_types=scratch,
    )
    def k(msg_hbm, dst_hbm, z_hbm, *refs):
        (sum_out, idx_v, rows_v, acc_s, sem_i, sem_m, sem_s) = refs
        cid = lax.axis_index("c")
        sid = lax.axis_index("s")
        wid = sid * NC + cid
        nch = (nct - 1 - wid) // NW + 1

        def fire_idx(j, c):
            pltpu.async_copy(dst_hbm.at[c0 + wid + j * NW], idx_v.at[j],
                             sem_i)
            return c
        lax.fori_loop(0, nch, fire_idx, 0)

        @pl.when(sid == 0)
        def _():
            pltpu.sync_copy(z_hbm, acc_s)

        def drain_idx(j, c):
            pltpu.make_async_copy(dst_hbm.at[c0], idx_v.at[0], sem_i).wait()
            return c
        lax.fori_loop(0, nch, drain_idx, 0)
        plsc.subcore_barrier()

        def fire_msg(j):
            base = (wid + j * NW) * CH
            pltpu.async_copy(msg_hbm.at[pl.ds(base, CH)],
                             rows_v.at[lax.rem(j, 2)], sem_m)

        fire_msg(0)

        def body(j, carry):
            @pl.when(j + 1 < nch)
            def _():
                @pl.when(j >= 1)
                def _():
                    pltpu.make_async_copy(rows_v.at[lax.rem(j - 1, 2)],
                                          acc_s.at[idx_v.at[j - 1]],
                                          sem_s).wait()
                fire_msg(j + 1)
            base = (wid + j * NW) * CH
            pltpu.make_async_copy(msg_hbm.at[pl.ds(base, CH)],
                                  rows_v.at[lax.rem(j, 2)], sem_m).wait()
            pltpu.async_copy(rows_v.at[lax.rem(j, 2)], acc_s.at[idx_v.at[j]],
                             sem_s, add=True)
            return carry

        lax.fori_loop(0, nch, body, 0)

        def drain_s(j, c):
            pltpu.make_async_copy(rows_v.at[lax.rem(j, 2)],
                                  acc_s.at[idx_v.at[j]], sem_s).wait()
            return c
        lax.fori_loop(jnp.maximum(nch - 2, 0), nch, drain_s, 0)
        plsc.subcore_barrier()

        r0 = sid * rows_per_tile
        pltpu.sync_copy(acc_s.at[pl.ds(r0, rows_per_tile)],
                        sum_out.at[cid].at[pl.ds(r0, rows_per_tile)])

    return k(msg, dst2, zeros_nc)[0]


def _count_edges(dst2, n, zeros_cnt):
    n_chunks = dst2.shape[0]
    rows_per_tile = n // NS
    maxc = (n_chunks + NW - 1) // NW

    @functools.partial(
        pl.kernel,
        out_type=jax.ShapeDtypeStruct((NC, n, CW), jnp.float32),
        mesh=_sc_mesh(),
        compiler_params=_SC_PARAMS,
        scratch_types=[
            pltpu.VMEM((maxc, CH), jnp.int32),
            pltpu.VMEM((CH, CW), jnp.float32),
            pltpu.VMEM_SHARED((n, CW), jnp.float32),
            pltpu.SemaphoreType.DMA,
        ],
    )
    def k(dst_hbm, zc_hbm, cnt_out, idx_v, ones_v, cnt_s, sem_i):
        cid = lax.axis_index("c")
        sid = lax.axis_index("s")
        wid = sid * NC + cid
        nch = (n_chunks - 1 - wid) // NW + 1

        def fire_idx(j, c):
            pltpu.async_copy(dst_hbm.at[wid + j * NW], idx_v.at[j], sem_i)
            return c
        lax.fori_loop(0, nch, fire_idx, 0)

        @pl.when(sid == 0)
        def _():
            pltpu.sync_copy(zc_hbm, cnt_s)

        def fill(i, c):
            ones_v[i, :] = jnp.ones((CW,), jnp.float32)
            return c
        lax.fori_loop(0, CH, fill, 0)

        def drain_idx(j, c):
            pltpu.make_async_copy(dst_hbm.at[wid], idx_v.at[0], sem_i).wait()
            return c
        lax.fori_loop(0, nch, drain_idx, 0)
        plsc.subcore_barrier()

        def body(j, c):
            pltpu.sync_copy(ones_v, cnt_s.at[idx_v.at[j]], add=True)
            return c
        lax.fori_loop(0, nch, body, 0)
        plsc.subcore_barrier()

        r0 = sid * rows_per_tile
        pltpu.sync_copy(cnt_s.at[pl.ds(r0, rows_per_tile)],
                        cnt_out.at[cid].at[pl.ds(r0, rows_per_tile)])

    return k(dst2, zeros_cnt)


def _edge_hidden(ea4, w1bd1, b1t1, w1bd2, b1t2):
    rows = ea4.shape[0]
    RB = 4000
    grid = rows // RB

    def body(ea_ref, w1_ref, b1_ref, w2_ref, b2_ref, o_ref):
        ea = ea_ref[...]
        h1 = jnp.maximum(
            jnp.dot(ea, w1_ref[...], preferred_element_type=jnp.float32)
            + b1_ref[...], 0.0)
        h2 = jnp.maximum(
            jnp.dot(ea, w2_ref[...], preferred_element_type=jnp.float32)
            + b2_ref[...], 0.0)
        o_ref[...] = jnp.concatenate([h1, h2], axis=1)

    full = lambda s: pl.BlockSpec(s, lambda i: (0, 0))
    return pl.pallas_call(
        body,
        grid=(grid,),
        in_specs=[
            pl.BlockSpec((RB, 16), lambda i: (i, 0)),
            full((16, 4 * KH)), full((1, 4 * KH)),
            full((16, 4 * KH)), full((1, 4 * KH)),
        ],
        out_specs=pl.BlockSpec((RB, 8 * KH), lambda i: (i, 0)),
        out_shape=jax.ShapeDtypeStruct((rows, 8 * KH), jnp.float32),
    )(ea4, w1bd1, b1t1, w1bd2, b1t2)


def _edge_messages(hcombo, xj128, layer, roff, R, W2r, B2r):
    rows = xj128.shape[0]
    RB = 4000
    grid = rows // RB
    hoff = layer * 4 * KH

    def body(h_ref, xj_ref, R_ref, W2_ref, B2_ref, o_ref):
        xj = xj_ref[...]
        parts = []
        for j in range(4):
            h_j = h_ref[:, hoff + j * KH:hoff + (j + 1) * KH]
            x_j = xj[:, j * IC:(j + 1) * IC]
            A = jnp.dot(h_j, R_ref[...], preferred_element_type=jnp.float32)
            B = jnp.tile(x_j, (1, KH))
            acc = jnp.dot(A * B, W2_ref[...],
                          preferred_element_type=jnp.float32)
            acc = acc + jnp.dot(x_j, B2_ref[...],
                                preferred_element_type=jnp.float32)
            parts.append(acc)
        o_ref[...] = jnp.concatenate(parts, axis=1)

    full = lambda s: pl.BlockSpec(s, lambda i: (0, 0))
    return pl.pallas_call(
        body,
        grid=(grid,),
        in_specs=[
            pl.BlockSpec((RB, 8 * KH), lambda i: (i + roff, 0)),
            pl.BlockSpec((RB, 4 * IC), lambda i: (i, 0)),
            full((KH, KH * IC)),
            full((KH * IC, OC)), full((IC, OC)),
        ],
        out_specs=pl.BlockSpec((RB, 4 * OC), lambda i: (i, 0)),
        out_shape=jax.ShapeDtypeStruct((rows, 4 * OC), jnp.float32),
    )(hcombo, xj128, R, W2r, B2r)


def _combine1(sums_a, sums_b, cnts, x, root1, b1r):
    n = x.shape[0]

    def body(sa_ref, sb_ref, c_ref, x_ref, r_ref, b_ref, o_ref, inv_ref):
        cnt = c_ref[0] + c_ref[1]
        inv = 1.0 / jnp.maximum(cnt[:, 0:1], 1.0)
        mean = (sa_ref[0] + sa_ref[1] + sb_ref[0] + sb_ref[1]) * inv
        o_ref[...] = jnp.maximum(
            mean + jnp.dot(x_ref[...], r_ref[...],
                           preferred_element_type=jnp.float32) + b_ref[...],
            0.0)
        inv_ref[...] = inv

    return pl.pallas_call(
        body,
        out_shape=(jax.ShapeDtypeStruct((n, OC), jnp.float32),
                   jax.ShapeDtypeStruct((n, 1), jnp.float32)),
    )(sums_a, sums_b, cnts, x, root1, b1r)


def _combine2(sums_a, sums_b, inv, h1, root2, b2r, fc1_w, fc1_br, fc2_w,
              fc2_br):
    n = h1.shape[0]

    def body(sa_ref, sb_ref, i_ref, h_ref, r_ref, b_ref, f1_ref, f1b_ref,
             f2_ref, f2b_ref, o_ref):
        mean = (sa_ref[0] + sa_ref[1] + sb_ref[0] + sb_ref[1]) * i_ref[...]
        h2 = jnp.maximum(
            mean + jnp.dot(h_ref[...], r_ref[...],
                           preferred_element_type=jnp.float32) + b_ref[...],
            0.0)
        h3 = jnp.maximum(
            jnp.dot(h2, f1_ref[...], preferred_element_type=jnp.float32)
            + f1b_ref[...], 0.0)
        o_ref[...] = (jnp.dot(h3, f2_ref[...],
                              preferred_element_type=jnp.float32)
                      + f2b_ref[...])

    return pl.pallas_call(
        body,
        out_shape=jax.ShapeDtypeStruct((n, 1), jnp.float32),
    )(sums_a, sums_b, inv, h1, root2, b2r, fc1_w, fc1_br, fc2_w, fc2_br)


def _prep_w2(w2, b2, in_ch):
    W = w2.reshape(KH, in_ch, OC)
    B = b2.reshape(in_ch, OC)
    if in_ch < IC:
        W = jnp.pad(W, ((0, 0), (0, IC - in_ch), (0, 0)))
        B = jnp.pad(B, ((0, IC - in_ch), (0, 0)))
    return W.reshape(KH * IC, OC), B


def kernel(x, edge_index, edge_attr, nn1_w1, nn1_b1, nn1_w2, nn1_b2, root1,
           bias1, nn2_w1, nn2_b1, nn2_w2, nn2_b2, root2, bias2, fc1_w, fc1_b,
           fc2_w, fc2_b):
    n = x.shape[0]
    src = edge_index[0].astype(jnp.int32)
    dst = edge_index[1].astype(jnp.int32)

    R = jnp.repeat(jnp.eye(KH, dtype=jnp.float32), IC, axis=1)
    W2r1, B2r1 = _prep_w2(nn1_w2, nn1_b2, 3)
    W2r2, B2r2 = _prep_w2(nn2_w2, nn2_b2, IC)

    x_pad = jnp.pad(x, ((0, 0), (0, IC - x.shape[1])))
    zeros_nc = jnp.zeros((n, OC), jnp.float32)
    zeros_cnt = jnp.zeros((n, CW), jnp.float32)

    E = src.shape[0]
    src2 = src.reshape(E // CH, CH)
    dst2 = dst.reshape(E // CH, CH)

    ea4 = edge_attr.reshape(E // 4, 16)
    zblk = jnp.zeros((4, KH), jnp.float32)
    w1bd1 = jnp.block([[nn1_w1 if i == j else zblk for j in range(4)]
                       for i in range(4)])
    w1bd2 = jnp.block([[nn2_w1 if i == j else zblk for j in range(4)]
                       for i in range(4)])
    hcombo = _edge_hidden(ea4, w1bd1, jnp.tile(nn1_b1, 4).reshape(1, 4 * KH),
                          w1bd2, jnp.tile(nn2_b1, 4).reshape(1, 4 * KH))

    NCHK = E // CH
    HC = NCHK // 2
    EH = HC * CH
    RBH = EH // (4 * 4000)

    cnts = _count_edges(dst2, n, zeros_cnt)

    def layer(table, layer_idx, W2r, B2r):
        sums = []
        for half in range(2):
            xj = _gather_rows(table, src2, half * HC, HC)
            msg = _edge_messages(hcombo, xj.reshape(EH // 4, 4 * IC),
                                 layer_idx, half * RBH, R, W2r, B2r)
            sums.append(_scatter_add(msg.reshape(EH, OC), dst2, half * HC,
                                     HC, n, zeros_nc))
        return sums

    s1a, s1b = layer(x_pad, 0, W2r1, B2r1)
    h1, inv_cnt = _combine1(s1a, s1b, cnts, x, root1, bias1.reshape(1, OC))
    s2a, s2b = layer(h1, 1, W2r2, B2r2)
    out = _combine2(s2a, s2b, inv_cnt, h1, root2, bias2.reshape(1, OC),
                    fc1_w, fc1_b.reshape(1, OC), fc2_w, fc2_b.reshape(1, 1))
    return out
